# baseline (speedup 1.0000x reference)
import functools

import jax
import jax.numpy as jnp
from jax import lax
from jax.experimental import pallas as pl
from jax.experimental.pallas import tpu as pltpu
from jax.experimental.pallas import tpu_sc as plsc

N = 10000
E = 320000
F_IN = 128
EMB = 64

NC = 2
NS = 16
L = 16
NW = NC * NS
EPW = E // NW
C = 400
NCHUNK = EPW // C
NPAD = 10240
ROWS_PT = NPAD // NS

NR = 2
NI = 3

_DNUMS = lax.GatherDimensionNumbers(
    offset_dims=(), collapsed_slice_dims=(0,), start_index_map=(0,))


def _scale_rows_by_weight(rows_v, w_v):
    def group_body(g, _):
        wv = w_v[pl.ds(g * L, L)]
        for r in range(L):
            idx = jnp.full((L, 1), r, jnp.int32)
            wr = lax.gather(wv, idx, dimension_numbers=_DNUMS, slice_sizes=(1,),
                            mode=lax.GatherScatterMode.PROMISE_IN_BOUNDS)
            row = g * L + r
            for j in range(EMB // L):
                sl = pl.ds(j * L, L)
                rows_v[row, sl] = rows_v[row, sl] * wr
        return 0

    lax.fori_loop(0, C // L, group_body, 0)


def _make_sc_agg(with_cnt):
    mesh = plsc.VectorSubcoreMesh(core_axis_name="c", subcore_axis_name="s")

    out_type = [jax.ShapeDtypeStruct((NC * NPAD, EMB), jnp.float32)]
    if with_cnt:
        out_type.append(jax.ShapeDtypeStruct((NC * NPAD, L), jnp.float32))

    scratch = [
        [pltpu.VMEM((C,), jnp.int32) for _ in range(NI)],
        [pltpu.VMEM((C,), jnp.int32) for _ in range(NI)],
        [pltpu.VMEM((C,), jnp.float32) for _ in range(NI)],
        [pltpu.VMEM((C, EMB), jnp.float32) for _ in range(NR)],
        [pltpu.SemaphoreType.DMA for _ in range(NI)],
        [pltpu.SemaphoreType.DMA for _ in range(NR)],
        [pltpu.SemaphoreType.DMA for _ in range(NR)],
        pltpu.VMEM_SHARED((NPAD, EMB), jnp.float32),
    ]
    if with_cnt:
        scratch += [
            pltpu.VMEM((C, L), jnp.float32),
            pltpu.VMEM((C, L), jnp.float32),
            pltpu.VMEM_SHARED((NPAD, L), jnp.float32),
            pltpu.SemaphoreType.DMA,
        ]

    @functools.partial(
        pl.kernel,
        mesh=mesh,
        out_type=out_type,
        scratch_types=scratch,
        compiler_params=pltpu.CompilerParams(use_tc_tiling_on_sc=False),
    )
    def sc_agg(table_h, src_h, dst_h, w_h, *refs):
        if with_cnt:
            (featp_h, cntp_h, srcb, dstb, wb, rowsb, isem, gsem, ssem,
             accf, ones_v, z16_v, accc, csem) = refs
        else:
            (featp_h, srcb, dstb, wb, rowsb, isem, gsem, ssem, accf) = refs

        sid = lax.axis_index("s")
        cid = lax.axis_index("c")
        wid = cid * NS + sid
        base0 = wid * EPW

        def issue_idx(k, i):
            base = base0 + k * C
            pltpu.async_copy(src_h.at[pl.ds(base, C)], srcb[i], isem[i])
            pltpu.async_copy(dst_h.at[pl.ds(base, C)], dstb[i], isem[i])
            pltpu.async_copy(w_h.at[pl.ds(base, C)], wb[i], isem[i])

        def wait_idx(i):
            pltpu.make_async_copy(src_h.at[pl.ds(0, C)], srcb[i], isem[i]).wait()
            pltpu.make_async_copy(dst_h.at[pl.ds(0, C)], dstb[i], isem[i]).wait()
            pltpu.make_async_copy(w_h.at[pl.ds(0, C)], wb[i], isem[i]).wait()

        NSPLIT = 5
        CS = C // NSPLIT

        def issue_gather(i, b):
            for s in range(NSPLIT):
                pltpu.async_copy(
                    table_h.at[srcb[i].at[pl.ds(s * CS, CS)]],
                    rowsb[b].at[pl.ds(s * CS, CS)], gsem[b])

        def wait_gather(b):
            for s in range(NSPLIT):
                pltpu.make_async_copy(
                    table_h.at[srcb[0].at[pl.ds(0, CS)]],
                    rowsb[b].at[pl.ds(s * CS, CS)], gsem[b]).wait()

        def issue_scatter(i, b):
            pltpu.async_copy(rowsb[b], accf.at[dstb[i]], ssem[b], add=True)

        def wait_scatter(b):
            pltpu.make_async_copy(rowsb[b], accf.at[dstb[0]], ssem[b]).wait()

        def issue_cnt(i):
            pltpu.async_copy(ones_v, accc.at[dstb[i]], csem, add=True)

        def wait_cnt():
            pltpu.make_async_copy(ones_v, accc.at[dstb[0]], csem).wait()

        zv = jnp.zeros((L,), jnp.float32)
        ov = jnp.ones((L,), jnp.float32)

        def zero_body(i, _):
            for j in range(EMB // L):
                rowsb[0][i, pl.ds(j * L, L)] = zv
            if with_cnt:
                z16_v[i, :] = zv
                ones_v[i, :] = ov
            return 0

        lax.fori_loop(0, C, zero_body, 0)

        row0 = sid * ROWS_PT
        pltpu.sync_copy(rowsb[0].at[pl.ds(0, C)], accf.at[pl.ds(row0, C)])
        pltpu.sync_copy(rowsb[0].at[pl.ds(0, ROWS_PT - C)],
                        accf.at[pl.ds(row0 + C, ROWS_PT - C)])
        if with_cnt:
            pltpu.sync_copy(z16_v.at[pl.ds(0, C)], accc.at[pl.ds(row0, C)])
            pltpu.sync_copy(z16_v.at[pl.ds(0, ROWS_PT - C)],
                            accc.at[pl.ds(row0 + C, ROWS_PT - C)])

        issue_idx(0, 0)
        issue_idx(1, 1)
        wait_idx(0)
        issue_gather(0, 0)
        plsc.subcore_barrier()

        def body(k, phase, first):
            b = phase % NR
            o = (phase + 1) % NR
            i = phase % NI
            i1 = (phase + 1) % NI
            i2 = (phase + 2) % NI
            kt = k
            wait_gather(b)
            if not first:
                wait_scatter(o)

            @pl.when(kt <= NCHUNK - 2)
            def _():
                wait_idx(i1)
                issue_gather(i1, o)

            @pl.when(kt <= NCHUNK - 3)
            def _():
                issue_idx(k + 2, i2)

            if with_cnt:
                if not first:
                    wait_cnt()
                issue_cnt(i)
            _scale_rows_by_weight(rowsb[b], wb[i])
            issue_scatter(i, b)

        body(0, 0, first=True)

        def steady(t, _):
            k0 = 1 + t * 6
            for d in range(6):
                body(k0 + d, (1 + d) % 6, first=False)
            return 0

        lax.fori_loop(0, (NCHUNK - 1) // 6, steady, 0)

        wait_scatter((NCHUNK - 1) % NR)
        if with_cnt:
            wait_cnt()

        plsc.subcore_barrier()
        out0 = cid * NPAD + row0
        pltpu.sync_copy(accf.at[pl.ds(row0, ROWS_PT)],
                        featp_h.at[pl.ds(out0, ROWS_PT)])
        if with_cnt:
            pltpu.sync_copy(accc.at[pl.ds(row0, ROWS_PT)],
                            cntp_h.at[pl.ds(out0, ROWS_PT)])

    return sc_agg


_sc_agg_cnt = _make_sc_agg(with_cnt=True)
_sc_agg = _make_sc_agg(with_cnt=False)


def _tc_proj_body(x_ref, wl_ref, wr_ref, o1_ref, o2_ref):
    x = x_ref[...]
    o1_ref[...] = jnp.dot(x, wl_ref[...], preferred_element_type=jnp.float32)
    o2_ref[...] = jnp.dot(x, wr_ref[...], preferred_element_type=jnp.float32)


def _tc_proj(x, wlT, wrT):
    return pl.pallas_call(
        _tc_proj_body,
        out_shape=[
            jax.ShapeDtypeStruct((N, EMB), jnp.float32),
            jax.ShapeDtypeStruct((N, EMB), jnp.float32),
        ],
    )(x, wlT, wrT)


def _norm_relu(pre, gw, gb, gms):
    mean = jnp.mean(pre, axis=0, keepdims=True)
    cen = pre - gms * mean
    var = jnp.mean(cen * cen, axis=0, keepdims=True)
    return jnp.maximum(cen * jax.lax.rsqrt(var + 1e-5) * gw + gb, 0.0)


def _tc_mid(f1, c1, xr1, bl, gw, gb, gms, w2lT, w2rT):
    def body(f_ref, c_ref, xr_ref, bl_ref, gw_ref, gb_ref, gms_ref,
             w2l_ref, w2r_ref, h1_ref, hw_ref, hr_ref):
        cnt = c_ref[0:N, 0:1] + c_ref[NPAD:NPAD + N, 0:1]
        agg = (f_ref[0:N, :] + f_ref[NPAD:NPAD + N, :]) / jnp.maximum(cnt, 1.0)
        pre = agg + bl_ref[...] + xr_ref[...]
        h1 = _norm_relu(pre, gw_ref[...], gb_ref[...], gms_ref[...])
        h1_ref[...] = h1
        hw_ref[...] = jnp.dot(h1, w2l_ref[...], preferred_element_type=jnp.float32)
        hr_ref[...] = jnp.dot(h1, w2r_ref[...], preferred_element_type=jnp.float32)

    return pl.pallas_call(
        body,
        out_shape=[
            jax.ShapeDtypeStruct((N, EMB), jnp.float32),
            jax.ShapeDtypeStruct((N, EMB), jnp.float32),
            jax.ShapeDtypeStruct((N, EMB), jnp.float32),
        ],
    )(f1, c1, xr1, bl, gw, gb, gms, w2lT, w2rT)


def _sigmoid(x):
    return 1.0 / (1.0 + jnp.exp(-x))


def _tc_final(f2, c1, hr2, bl, gw, gb, gms, h1a,
              wih1T, bih1, bhh1, wih2T, bih2, bhh2):
    def body(f_ref, c_ref, hr_ref, bl_ref, gw_ref, gb_ref, gms_ref, h1a_ref,
             wih1_ref, bih1_ref, bhh1_ref, wih2_ref, bih2_ref, bhh2_ref,
             xc_ref, h1o_ref, h2o_ref):
        cnt = c_ref[0:N, 0:1] + c_ref[NPAD:NPAD + N, 0:1]
        agg = (f_ref[0:N, :] + f_ref[NPAD:NPAD + N, :]) / jnp.maximum(cnt, 1.0)
        pre = agg + bl_ref[...] + hr_ref[...]
        h2a = _norm_relu(pre, gw_ref[...], gb_ref[...], gms_ref[...])
        xc = jnp.concatenate([h1a_ref[...], h2a], axis=1)
        xc_ref[...] = xc

        gi = jnp.dot(xc, wih1_ref[...], preferred_element_type=jnp.float32) + bih1_ref[...]
        bhh = bhh1_ref[...]
        r = _sigmoid(gi[:, :EMB] + bhh[:, :EMB])
        z = _sigmoid(gi[:, EMB:2 * EMB] + bhh[:, EMB:2 * EMB])
        n = jnp.tanh(gi[:, 2 * EMB:] + r * bhh[:, 2 * EMB:])
        h1g = (1.0 - z) * n
        h1o_ref[...] = h1g

        gi2 = jnp.dot(h1g, wih2_ref[...], preferred_element_type=jnp.float32) + bih2_ref[...]
        bhh2v = bhh2_ref[...]
        r2 = _sigmoid(gi2[:, :EMB] + bhh2v[:, :EMB])
        z2 = _sigmoid(gi2[:, EMB:2 * EMB] + bhh2v[:, EMB:2 * EMB])
        n2 = jnp.tanh(gi2[:, 2 * EMB:] + r2 * bhh2v[:, 2 * EMB:])
        h2o_ref[...] = (1.0 - z2) * n2

    return pl.pallas_call(
        body,
        out_shape=[
            jax.ShapeDtypeStruct((N, 2 * EMB), jnp.float32),
            jax.ShapeDtypeStruct((N, EMB), jnp.float32),
            jax.ShapeDtypeStruct((N, EMB), jnp.float32),
        ],
    )(f2, c1, hr2, bl, gw, gb, gms, h1a, wih1T, bih1, bhh1, wih2T, bih2, bhh2)


def kernel(x, edge_index, edge_attr, W_l1, b_l1, W_r1, W_l2, b_l2, W_r2,
           gn1_w, gn1_b, gn1_ms, gn2_w, gn2_b, gn2_ms,
           W_ih1, W_hh1, b_ih1, b_hh1, W_ih2, W_hh2, b_ih2, b_hh2):
    src = edge_index[0]
    dst = edge_index[1]
    w = edge_attr[:, 0]

    xW1, xr1 = _tc_proj(x, W_l1.T, W_r1.T)

    f1, c1 = _sc_agg_cnt(xW1, src, dst, w)

    h1a, hW2, hr2 = _tc_mid(
        f1, c1, xr1, b_l1.reshape(1, EMB),
        gn1_w.reshape(1, EMB), gn1_b.reshape(1, EMB), gn1_ms.reshape(1, EMB),
        W_l2.T, W_r2.T)

    (f2,) = _sc_agg(hW2, src, dst, w)

    xc, h_1, h_2 = _tc_final(
        f2, c1, hr2, b_l2.reshape(1, EMB),
        gn2_w.reshape(1, EMB), gn2_b.reshape(1, EMB), gn2_ms.reshape(1, EMB),
        h1a, W_ih1.T, b_ih1.reshape(1, 3 * EMB), b_hh1.reshape(1, 3 * EMB),
        W_ih2.T, b_ih2.reshape(1, 3 * EMB), b_hh2.reshape(1, 3 * EMB))

    return (xc, h_1, h_2)


# speedup vs baseline: 1.8511x; 1.8511x over previous
import functools

import jax
import jax.numpy as jnp
from jax import lax
from jax.experimental import pallas as pl
from jax.experimental.pallas import tpu as pltpu
from jax.experimental.pallas import tpu_sc as plsc

N = 10000
E = 320000
F_IN = 128
EMB = 64

NC = 2
NS = 16
L = 16
NW = NC * NS
EPW = E // NW
C = 400
NCHUNK = EPW // C
NPAD = 10240
ROWS_PT = NPAD // NS

NR = 2
NI = 3

_DNUMS = lax.GatherDimensionNumbers(
    offset_dims=(), collapsed_slice_dims=(0,), start_index_map=(0,))


def _scale_rows_by_weight(rows_v, w_v):
    @plsc.parallel_loop(0, C // L, 1, unroll=2)
    def group_body(g):
        wv = w_v[pl.ds(g * L, L)]
        for r in range(L):
            idx = jnp.full((L, 1), r, jnp.int32)
            wr = lax.gather(wv, idx, dimension_numbers=_DNUMS, slice_sizes=(1,),
                            mode=lax.GatherScatterMode.PROMISE_IN_BOUNDS)
            row = g * L + r
            for j in range(EMB // L):
                sl = pl.ds(j * L, L)
                rows_v[row, sl] = rows_v[row, sl] * wr


def _make_sc_agg(with_cnt):
    mesh = plsc.VectorSubcoreMesh(core_axis_name="c", subcore_axis_name="s")

    out_type = [jax.ShapeDtypeStruct((NC * NPAD, EMB), jnp.float32)]
    if with_cnt:
        out_type.append(jax.ShapeDtypeStruct((NC * NPAD, L), jnp.float32))

    scratch = [
        [pltpu.VMEM((C,), jnp.int32) for _ in range(NI)],
        [pltpu.VMEM((C,), jnp.int32) for _ in range(NI)],
        [pltpu.VMEM((C,), jnp.float32) for _ in range(NI)],
        [pltpu.VMEM((C, EMB), jnp.float32) for _ in range(NR)],
        [pltpu.SemaphoreType.DMA for _ in range(NI)],
        [pltpu.SemaphoreType.DMA for _ in range(NR)],
        [pltpu.SemaphoreType.DMA for _ in range(NR)],
        pltpu.VMEM_SHARED((NPAD, EMB), jnp.float32),
    ]
    if with_cnt:
        scratch += [
            pltpu.VMEM((C, L), jnp.float32),
            pltpu.VMEM((C, L), jnp.float32),
            pltpu.VMEM_SHARED((NPAD, L), jnp.float32),
            pltpu.SemaphoreType.DMA,
        ]

    @functools.partial(
        pl.kernel,
        mesh=mesh,
        out_type=out_type,
        scratch_types=scratch,
        compiler_params=pltpu.CompilerParams(use_tc_tiling_on_sc=False),
    )
    def sc_agg(table_h, src_h, dst_h, w_h, *refs):
        if with_cnt:
            (featp_h, cntp_h, srcb, dstb, wb, rowsb, isem, gsem, ssem,
             accf, ones_v, z16_v, accc, csem) = refs
        else:
            (featp_h, srcb, dstb, wb, rowsb, isem, gsem, ssem, accf) = refs

        sid = lax.axis_index("s")
        cid = lax.axis_index("c")
        wid = cid * NS + sid
        base0 = wid * EPW

        def issue_idx(k, i):
            base = base0 + k * C
            pltpu.async_copy(src_h.at[pl.ds(base, C)], srcb[i], isem[i])
            pltpu.async_copy(dst_h.at[pl.ds(base, C)], dstb[i], isem[i])
            pltpu.async_copy(w_h.at[pl.ds(base, C)], wb[i], isem[i])

        def wait_idx(i):
            pltpu.make_async_copy(src_h.at[pl.ds(0, C)], srcb[i], isem[i]).wait()
            pltpu.make_async_copy(dst_h.at[pl.ds(0, C)], dstb[i], isem[i]).wait()
            pltpu.make_async_copy(w_h.at[pl.ds(0, C)], wb[i], isem[i]).wait()

        NSPLIT = 5
        CS = C // NSPLIT

        def issue_gather(i, b):
            for s in range(NSPLIT):
                pltpu.async_copy(
                    table_h.at[srcb[i].at[pl.ds(s * CS, CS)]],
                    rowsb[b].at[pl.ds(s * CS, CS)], gsem[b])

        def wait_gather(b):
            for s in range(NSPLIT):
                pltpu.make_async_copy(
                    table_h.at[srcb[0].at[pl.ds(0, CS)]],
                    rowsb[b].at[pl.ds(s * CS, CS)], gsem[b]).wait()

        def issue_scatter(i, b):
            pltpu.async_copy(rowsb[b], accf.at[dstb[i]], ssem[b], add=True)

        def wait_scatter(b):
            pltpu.make_async_copy(rowsb[b], accf.at[dstb[0]], ssem[b]).wait()

        def issue_cnt(i):
            pltpu.async_copy(ones_v, accc.at[dstb[i]], csem, add=True)

        def wait_cnt():
            pltpu.make_async_copy(ones_v, accc.at[dstb[0]], csem).wait()

        zv = jnp.zeros((L,), jnp.float32)
        ov = jnp.ones((L,), jnp.float32)

        def zero_body(i, _):
            for j in range(EMB // L):
                rowsb[0][i, pl.ds(j * L, L)] = zv
            if with_cnt:
                z16_v[i, :] = zv
                ones_v[i, :] = ov
            return 0

        lax.fori_loop(0, C, zero_body, 0)

        row0 = sid * ROWS_PT
        pltpu.sync_copy(rowsb[0].at[pl.ds(0, C)], accf.at[pl.ds(row0, C)])
        pltpu.sync_copy(rowsb[0].at[pl.ds(0, ROWS_PT - C)],
                        accf.at[pl.ds(row0 + C, ROWS_PT - C)])
        if with_cnt:
            pltpu.sync_copy(z16_v.at[pl.ds(0, C)], accc.at[pl.ds(row0, C)])
            pltpu.sync_copy(z16_v.at[pl.ds(0, ROWS_PT - C)],
                            accc.at[pl.ds(row0 + C, ROWS_PT - C)])

        issue_idx(0, 0)
        issue_idx(1, 1)
        wait_idx(0)
        issue_gather(0, 0)
        plsc.subcore_barrier()

        def body(k, phase, first):
            b = phase % NR
            o = (phase + 1) % NR
            i = phase % NI
            i1 = (phase + 1) % NI
            i2 = (phase + 2) % NI
            kt = k
            wait_gather(b)
            if not first:
                wait_scatter(o)

            @pl.when(kt <= NCHUNK - 2)
            def _():
                wait_idx(i1)
                issue_gather(i1, o)

            @pl.when(kt <= NCHUNK - 3)
            def _():
                issue_idx(k + 2, i2)

            if with_cnt:
                if not first:
                    wait_cnt()
                issue_cnt(i)
            _scale_rows_by_weight(rowsb[b], wb[i])
            issue_scatter(i, b)

        body(0, 0, first=True)

        def steady(t, _):
            k0 = 1 + t * 6
            for d in range(6):
                body(k0 + d, (1 + d) % 6, first=False)
            return 0

        lax.fori_loop(0, (NCHUNK - 1) // 6, steady, 0)

        wait_scatter((NCHUNK - 1) % NR)
        if with_cnt:
            wait_cnt()

        plsc.subcore_barrier()
        out0 = cid * NPAD + row0
        pltpu.sync_copy(accf.at[pl.ds(row0, ROWS_PT)],
                        featp_h.at[pl.ds(out0, ROWS_PT)])
        if with_cnt:
            pltpu.sync_copy(accc.at[pl.ds(row0, ROWS_PT)],
                            cntp_h.at[pl.ds(out0, ROWS_PT)])

    return sc_agg


_sc_agg_cnt = _make_sc_agg(with_cnt=True)
_sc_agg = _make_sc_agg(with_cnt=False)


def _tc_proj_body(x_ref, wl_ref, wr_ref, o1_ref, o2_ref):
    x = x_ref[...]
    o1_ref[...] = jnp.dot(x, wl_ref[...], preferred_element_type=jnp.float32)
    o2_ref[...] = jnp.dot(x, wr_ref[...], preferred_element_type=jnp.float32)


def _tc_proj(x, wlT, wrT):
    return pl.pallas_call(
        _tc_proj_body,
        out_shape=[
            jax.ShapeDtypeStruct((N, EMB), jnp.float32),
            jax.ShapeDtypeStruct((N, EMB), jnp.float32),
        ],
    )(x, wlT, wrT)


def _norm_relu(pre, gw, gb, gms):
    mean = jnp.mean(pre, axis=0, keepdims=True)
    cen = pre - gms * mean
    var = jnp.mean(cen * cen, axis=0, keepdims=True)
    return jnp.maximum(cen * jax.lax.rsqrt(var + 1e-5) * gw + gb, 0.0)


def _tc_mid(f1, c1, xr1, bl, gw, gb, gms, w2lT, w2rT):
    def body(f_ref, c_ref, xr_ref, bl_ref, gw_ref, gb_ref, gms_ref,
             w2l_ref, w2r_ref, h1_ref, hw_ref, hr_ref):
        cnt = c_ref[0:N, 0:1] + c_ref[NPAD:NPAD + N, 0:1]
        agg = (f_ref[0:N, :] + f_ref[NPAD:NPAD + N, :]) / jnp.maximum(cnt, 1.0)
        pre = agg + bl_ref[...] + xr_ref[...]
        h1 = _norm_relu(pre, gw_ref[...], gb_ref[...], gms_ref[...])
        h1_ref[...] = h1
        hw_ref[...] = jnp.dot(h1, w2l_ref[...], preferred_element_type=jnp.float32)
        hr_ref[...] = jnp.dot(h1, w2r_ref[...], preferred_element_type=jnp.float32)

    return pl.pallas_call(
        body,
        out_shape=[
            jax.ShapeDtypeStruct((N, EMB), jnp.float32),
            jax.ShapeDtypeStruct((N, EMB), jnp.float32),
            jax.ShapeDtypeStruct((N, EMB), jnp.float32),
        ],
    )(f1, c1, xr1, bl, gw, gb, gms, w2lT, w2rT)


def _sigmoid(x):
    return 1.0 / (1.0 + jnp.exp(-x))


def _tc_final(f2, c1, hr2, bl, gw, gb, gms, h1a,
              wih1T, bih1, bhh1, wih2T, bih2, bhh2):
    def body(f_ref, c_ref, hr_ref, bl_ref, gw_ref, gb_ref, gms_ref, h1a_ref,
             wih1_ref, bih1_ref, bhh1_ref, wih2_ref, bih2_ref, bhh2_ref,
             xc_ref, h1o_ref, h2o_ref):
        cnt = c_ref[0:N, 0:1] + c_ref[NPAD:NPAD + N, 0:1]
        agg = (f_ref[0:N, :] + f_ref[NPAD:NPAD + N, :]) / jnp.maximum(cnt, 1.0)
        pre = agg + bl_ref[...] + hr_ref[...]
        h2a = _norm_relu(pre, gw_ref[...], gb_ref[...], gms_ref[...])
        xc = jnp.concatenate([h1a_ref[...], h2a], axis=1)
        xc_ref[...] = xc

        gi = jnp.dot(xc, wih1_ref[...], preferred_element_type=jnp.float32) + bih1_ref[...]
        bhh = bhh1_ref[...]
        r = _sigmoid(gi[:, :EMB] + bhh[:, :EMB])
        z = _sigmoid(gi[:, EMB:2 * EMB] + bhh[:, EMB:2 * EMB])
        n = jnp.tanh(gi[:, 2 * EMB:] + r * bhh[:, 2 * EMB:])
        h1g = (1.0 - z) * n
        h1o_ref[...] = h1g

        gi2 = jnp.dot(h1g, wih2_ref[...], preferred_element_type=jnp.float32) + bih2_ref[...]
        bhh2v = bhh2_ref[...]
        r2 = _sigmoid(gi2[:, :EMB] + bhh2v[:, :EMB])
        z2 = _sigmoid(gi2[:, EMB:2 * EMB] + bhh2v[:, EMB:2 * EMB])
        n2 = jnp.tanh(gi2[:, 2 * EMB:] + r2 * bhh2v[:, 2 * EMB:])
        h2o_ref[...] = (1.0 - z2) * n2

    return pl.pallas_call(
        body,
        out_shape=[
            jax.ShapeDtypeStruct((N, 2 * EMB), jnp.float32),
            jax.ShapeDtypeStruct((N, EMB), jnp.float32),
            jax.ShapeDtypeStruct((N, EMB), jnp.float32),
        ],
    )(f2, c1, hr2, bl, gw, gb, gms, h1a, wih1T, bih1, bhh1, wih2T, bih2, bhh2)


def kernel(x, edge_index, edge_attr, W_l1, b_l1, W_r1, W_l2, b_l2, W_r2,
           gn1_w, gn1_b, gn1_ms, gn2_w, gn2_b, gn2_ms,
           W_ih1, W_hh1, b_ih1, b_hh1, W_ih2, W_hh2, b_ih2, b_hh2):
    src = edge_index[0]
    dst = edge_index[1]
    w = edge_attr[:, 0]

    xW1, xr1 = _tc_proj(x, W_l1.T, W_r1.T)

    f1, c1 = _sc_agg_cnt(xW1, src, dst, w)

    h1a, hW2, hr2 = _tc_mid(
        f1, c1, xr1, b_l1.reshape(1, EMB),
        gn1_w.reshape(1, EMB), gn1_b.reshape(1, EMB), gn1_ms.reshape(1, EMB),
        W_l2.T, W_r2.T)

    (f2,) = _sc_agg(hW2, src, dst, w)

    xc, h_1, h_2 = _tc_final(
        f2, c1, hr2, b_l2.reshape(1, EMB),
        gn2_w.reshape(1, EMB), gn2_b.reshape(1, EMB), gn2_ms.reshape(1, EMB),
        h1a, W_ih1.T, b_ih1.reshape(1, 3 * EMB), b_hh1.reshape(1, 3 * EMB),
        W_ih2.T, b_ih2.reshape(1, 3 * EMB), b_hh2.reshape(1, 3 * EMB))

    return (xc, h_1, h_2)


# speedup vs baseline: 1.9209x; 1.0377x over previous
import functools

import jax
import jax.numpy as jnp
from jax import lax
from jax.experimental import pallas as pl
from jax.experimental.pallas import tpu as pltpu
from jax.experimental.pallas import tpu_sc as plsc

N = 10000
E = 320000
F_IN = 128
EMB = 64

NC = 2
NS = 16
L = 16
NW = NC * NS
EPW = E // NW
C = 400
NCHUNK = EPW // C
NPAD = 10240
ROWS_PT = NPAD // NS

NR = 2
NI = 3

_DNUMS = lax.GatherDimensionNumbers(
    offset_dims=(), collapsed_slice_dims=(0,), start_index_map=(0,))


def _scale_rows_by_weight(rows_v, w_v):
    @plsc.parallel_loop(0, C // L, 1, unroll=2)
    def group_body(g):
        wv = w_v[pl.ds(g * L, L)]
        for r in range(L):
            idx = jnp.full((L, 1), r, jnp.int32)
            wr = lax.gather(wv, idx, dimension_numbers=_DNUMS, slice_sizes=(1,),
                            mode=lax.GatherScatterMode.PROMISE_IN_BOUNDS)
            row = g * L + r
            for j in range(EMB // L):
                sl = pl.ds(j * L, L)
                rows_v[row, sl] = rows_v[row, sl] * wr


def _make_sc_agg(with_cnt):
    mesh = plsc.VectorSubcoreMesh(core_axis_name="c", subcore_axis_name="s")

    out_type = [jax.ShapeDtypeStruct((NC * NPAD, EMB), jnp.float32)]
    if with_cnt:
        out_type.append(jax.ShapeDtypeStruct((NC * NPAD, L), jnp.float32))

    scratch = [
        [pltpu.VMEM((C,), jnp.int32) for _ in range(NI)],
        [pltpu.VMEM((C,), jnp.int32) for _ in range(NI)],
        [pltpu.VMEM((C,), jnp.float32) for _ in range(NI)],
        [pltpu.VMEM((C, EMB), jnp.float32) for _ in range(NR)],
        [pltpu.SemaphoreType.DMA for _ in range(NI)],
        [pltpu.SemaphoreType.DMA for _ in range(NR)],
        [pltpu.SemaphoreType.DMA for _ in range(NR)],
        pltpu.VMEM_SHARED((NPAD, EMB), jnp.float32),
    ]
    if with_cnt:
        scratch += [
            pltpu.VMEM((C, L), jnp.float32),
            pltpu.VMEM((C, L), jnp.float32),
            pltpu.VMEM_SHARED((NPAD, L), jnp.float32),
            pltpu.SemaphoreType.DMA,
        ]

    @functools.partial(
        pl.kernel,
        mesh=mesh,
        out_type=out_type,
        scratch_types=scratch,
        compiler_params=pltpu.CompilerParams(use_tc_tiling_on_sc=False),
    )
    def sc_agg(table_h, eidx_h, w_h, *refs):
        if with_cnt:
            (featp_h, cntp_h, srcb, dstb, wb, rowsb, isem, gsem, ssem,
             accf, ones_v, z16_v, accc, csem) = refs
        else:
            (featp_h, srcb, dstb, wb, rowsb, isem, gsem, ssem, accf) = refs

        sid = lax.axis_index("s")
        cid = lax.axis_index("c")
        wid = cid * NS + sid
        base0 = wid * EPW

        def issue_idx(k, i):
            base = base0 + k * C
            pltpu.async_copy(eidx_h.at[pl.ds(base, C)], srcb[i], isem[i])
            pltpu.async_copy(eidx_h.at[pl.ds(E + base, C)], dstb[i], isem[i])
            pltpu.async_copy(w_h.at[pl.ds(base, C)], wb[i], isem[i])

        def wait_idx(i):
            pltpu.make_async_copy(eidx_h.at[pl.ds(0, C)], srcb[i], isem[i]).wait()
            pltpu.make_async_copy(eidx_h.at[pl.ds(0, C)], dstb[i], isem[i]).wait()
            pltpu.make_async_copy(w_h.at[pl.ds(0, C)], wb[i], isem[i]).wait()

        NSPLIT = 5
        CS = C // NSPLIT

        def issue_gather(i, b):
            for s in range(NSPLIT):
                pltpu.async_copy(
                    table_h.at[srcb[i].at[pl.ds(s * CS, CS)]],
                    rowsb[b].at[pl.ds(s * CS, CS)], gsem[b])

        def wait_gather(b):
            for s in range(NSPLIT):
                pltpu.make_async_copy(
                    table_h.at[srcb[0].at[pl.ds(0, CS)]],
                    rowsb[b].at[pl.ds(s * CS, CS)], gsem[b]).wait()

        def issue_scatter(i, b):
            pltpu.async_copy(rowsb[b], accf.at[dstb[i]], ssem[b], add=True)

        def wait_scatter(b):
            pltpu.make_async_copy(rowsb[b], accf.at[dstb[0]], ssem[b]).wait()

        def issue_cnt(i):
            pltpu.async_copy(ones_v, accc.at[dstb[i]], csem, add=True)

        def wait_cnt():
            pltpu.make_async_copy(ones_v, accc.at[dstb[0]], csem).wait()

        zv = jnp.zeros((L,), jnp.float32)
        ov = jnp.ones((L,), jnp.float32)

        def zero_body(i, _):
            for j in range(EMB // L):
                rowsb[0][i, pl.ds(j * L, L)] = zv
            if with_cnt:
                z16_v[i, :] = zv
                ones_v[i, :] = ov
            return 0

        lax.fori_loop(0, C, zero_body, 0)

        row0 = sid * ROWS_PT
        pltpu.sync_copy(rowsb[0].at[pl.ds(0, C)], accf.at[pl.ds(row0, C)])
        pltpu.sync_copy(rowsb[0].at[pl.ds(0, ROWS_PT - C)],
                        accf.at[pl.ds(row0 + C, ROWS_PT - C)])
        if with_cnt:
            pltpu.sync_copy(z16_v.at[pl.ds(0, C)], accc.at[pl.ds(row0, C)])
            pltpu.sync_copy(z16_v.at[pl.ds(0, ROWS_PT - C)],
                            accc.at[pl.ds(row0 + C, ROWS_PT - C)])

        issue_idx(0, 0)
        issue_idx(1, 1)
        wait_idx(0)
        issue_gather(0, 0)
        plsc.subcore_barrier()

        def body(k, phase, first):
            b = phase % NR
            o = (phase + 1) % NR
            i = phase % NI
            i1 = (phase + 1) % NI
            i2 = (phase + 2) % NI
            kt = k
            wait_gather(b)
            if not first:
                wait_scatter(o)

            @pl.when(kt <= NCHUNK - 2)
            def _():
                wait_idx(i1)
                issue_gather(i1, o)

            @pl.when(kt <= NCHUNK - 3)
            def _():
                issue_idx(k + 2, i2)

            if with_cnt:
                if not first:
                    wait_cnt()
                issue_cnt(i)
            _scale_rows_by_weight(rowsb[b], wb[i])
            issue_scatter(i, b)

        body(0, 0, first=True)

        def steady(t, _):
            k0 = 1 + t * 6
            for d in range(6):
                body(k0 + d, (1 + d) % 6, first=False)
            return 0

        lax.fori_loop(0, (NCHUNK - 1) // 6, steady, 0)

        wait_scatter((NCHUNK - 1) % NR)
        if with_cnt:
            wait_cnt()

        plsc.subcore_barrier()
        out0 = cid * NPAD + row0
        pltpu.sync_copy(accf.at[pl.ds(row0, ROWS_PT)],
                        featp_h.at[pl.ds(out0, ROWS_PT)])
        if with_cnt:
            pltpu.sync_copy(accc.at[pl.ds(row0, ROWS_PT)],
                            cntp_h.at[pl.ds(out0, ROWS_PT)])

    return sc_agg


_sc_agg_cnt = _make_sc_agg(with_cnt=True)
_sc_agg = _make_sc_agg(with_cnt=False)


def _tc_proj_body(x_ref, wl_ref, wr_ref, o1_ref, o2_ref):
    x = x_ref[...]
    o1_ref[...] = jnp.dot(x, wl_ref[...], preferred_element_type=jnp.float32)
    o2_ref[...] = jnp.dot(x, wr_ref[...], preferred_element_type=jnp.float32)


def _tc_proj(x, wlT, wrT):
    return pl.pallas_call(
        _tc_proj_body,
        out_shape=[
            jax.ShapeDtypeStruct((N, EMB), jnp.float32),
            jax.ShapeDtypeStruct((N, EMB), jnp.float32),
        ],
    )(x, wlT, wrT)


def _norm_relu(pre, gw, gb, gms):
    mean = jnp.mean(pre, axis=0, keepdims=True)
    cen = pre - gms * mean
    var = jnp.mean(cen * cen, axis=0, keepdims=True)
    return jnp.maximum(cen * jax.lax.rsqrt(var + 1e-5) * gw + gb, 0.0)


def _tc_mid(f1, c1, xr1, bl, gw, gb, gms, w2lT, w2rT):
    def body(f_ref, c_ref, xr_ref, bl_ref, gw_ref, gb_ref, gms_ref,
             w2l_ref, w2r_ref, h1_ref, hw_ref, hr_ref):
        cnt = c_ref[0:N, 0:1] + c_ref[NPAD:NPAD + N, 0:1]
        agg = (f_ref[0:N, :] + f_ref[NPAD:NPAD + N, :]) / jnp.maximum(cnt, 1.0)
        pre = agg + bl_ref[...] + xr_ref[...]
        h1 = _norm_relu(pre, gw_ref[...], gb_ref[...], gms_ref[...])
        h1_ref[...] = h1
        hw_ref[...] = jnp.dot(h1, w2l_ref[...], preferred_element_type=jnp.float32)
        hr_ref[...] = jnp.dot(h1, w2r_ref[...], preferred_element_type=jnp.float32)

    return pl.pallas_call(
        body,
        out_shape=[
            jax.ShapeDtypeStruct((N, EMB), jnp.float32),
            jax.ShapeDtypeStruct((N, EMB), jnp.float32),
            jax.ShapeDtypeStruct((N, EMB), jnp.float32),
        ],
    )(f1, c1, xr1, bl, gw, gb, gms, w2lT, w2rT)


def _sigmoid(x):
    return 1.0 / (1.0 + jnp.exp(-x))


def _tc_final(f2, c1, hr2, bl, gw, gb, gms, h1a,
              wih1T, bih1, bhh1, wih2T, bih2, bhh2):
    def body(f_ref, c_ref, hr_ref, bl_ref, gw_ref, gb_ref, gms_ref, h1a_ref,
             wih1_ref, bih1_ref, bhh1_ref, wih2_ref, bih2_ref, bhh2_ref,
             xc_ref, h1o_ref, h2o_ref):
        cnt = c_ref[0:N, 0:1] + c_ref[NPAD:NPAD + N, 0:1]
        agg = (f_ref[0:N, :] + f_ref[NPAD:NPAD + N, :]) / jnp.maximum(cnt, 1.0)
        pre = agg + bl_ref[...] + hr_ref[...]
        h2a = _norm_relu(pre, gw_ref[...], gb_ref[...], gms_ref[...])
        xc = jnp.concatenate([h1a_ref[...], h2a], axis=1)
        xc_ref[...] = xc

        gi = jnp.dot(xc, wih1_ref[...], preferred_element_type=jnp.float32) + bih1_ref[...]
        bhh = bhh1_ref[...]
        r = _sigmoid(gi[:, :EMB] + bhh[:, :EMB])
        z = _sigmoid(gi[:, EMB:2 * EMB] + bhh[:, EMB:2 * EMB])
        n = jnp.tanh(gi[:, 2 * EMB:] + r * bhh[:, 2 * EMB:])
        h1g = (1.0 - z) * n
        h1o_ref[...] = h1g

        gi2 = jnp.dot(h1g, wih2_ref[...], preferred_element_type=jnp.float32) + bih2_ref[...]
        bhh2v = bhh2_ref[...]
        r2 = _sigmoid(gi2[:, :EMB] + bhh2v[:, :EMB])
        z2 = _sigmoid(gi2[:, EMB:2 * EMB] + bhh2v[:, EMB:2 * EMB])
        n2 = jnp.tanh(gi2[:, 2 * EMB:] + r2 * bhh2v[:, 2 * EMB:])
        h2o_ref[...] = (1.0 - z2) * n2

    return pl.pallas_call(
        body,
        out_shape=[
            jax.ShapeDtypeStruct((N, 2 * EMB), jnp.float32),
            jax.ShapeDtypeStruct((N, EMB), jnp.float32),
            jax.ShapeDtypeStruct((N, EMB), jnp.float32),
        ],
    )(f2, c1, hr2, bl, gw, gb, gms, h1a, wih1T, bih1, bhh1, wih2T, bih2, bhh2)


def kernel(x, edge_index, edge_attr, W_l1, b_l1, W_r1, W_l2, b_l2, W_r2,
           gn1_w, gn1_b, gn1_ms, gn2_w, gn2_b, gn2_ms,
           W_ih1, W_hh1, b_ih1, b_hh1, W_ih2, W_hh2, b_ih2, b_hh2):
    eidx = edge_index.reshape(2 * E)
    w = edge_attr.reshape(E)

    xW1, xr1 = _tc_proj(x, W_l1.T, W_r1.T)

    f1, c1 = _sc_agg_cnt(xW1, eidx, w)

    h1a, hW2, hr2 = _tc_mid(
        f1, c1, xr1, b_l1.reshape(1, EMB),
        gn1_w.reshape(1, EMB), gn1_b.reshape(1, EMB), gn1_ms.reshape(1, EMB),
        W_l2.T, W_r2.T)

    (f2,) = _sc_agg(hW2, eidx, w)

    xc, h_1, h_2 = _tc_final(
        f2, c1, hr2, b_l2.reshape(1, EMB),
        gn2_w.reshape(1, EMB), gn2_b.reshape(1, EMB), gn2_ms.reshape(1, EMB),
        h1a, W_ih1.T, b_ih1.reshape(1, 3 * EMB), b_hh1.reshape(1, 3 * EMB),
        W_ih2.T, b_ih2.reshape(1, 3 * EMB), b_hh2.reshape(1, 3 * EMB))

    return (xc, h_1, h_2)


# speedup vs baseline: 2.0381x; 1.0610x over previous
import functools

import jax
import jax.numpy as jnp
from jax import lax
from jax.experimental import pallas as pl
from jax.experimental.pallas import tpu as pltpu
from jax.experimental.pallas import tpu_sc as plsc

N = 10000
E = 320000
F_IN = 128
EMB = 64

NC = 2
NS = 16
L = 16
NW = NC * NS
EPW = E // NW
C = 400
NCHUNK = EPW // C
NPAD = 10240
ROWS_PT = NPAD // NS

NR = 2
NI = 3

_DNUMS = lax.GatherDimensionNumbers(
    offset_dims=(), collapsed_slice_dims=(0,), start_index_map=(0,))


def _scale_rows_by_weight(rows_v, w_v):
    @plsc.parallel_loop(0, C // L, 1, unroll=2)
    def group_body(g):
        wv = w_v[pl.ds(g * L, L)]
        for r in range(L):
            idx = jnp.full((L, 1), r, jnp.int32)
            wr = lax.gather(wv, idx, dimension_numbers=_DNUMS, slice_sizes=(1,),
                            mode=lax.GatherScatterMode.PROMISE_IN_BOUNDS)
            row = g * L + r
            for j in range(EMB // L):
                sl = pl.ds(j * L, L)
                rows_v[row, sl] = rows_v[row, sl] * wr


def _make_sc_agg(with_cnt):
    mesh = plsc.VectorSubcoreMesh(core_axis_name="c", subcore_axis_name="s")

    out_type = [jax.ShapeDtypeStruct((NC * NPAD, 2 * EMB), jnp.float32)]
    if with_cnt:
        out_type.append(jax.ShapeDtypeStruct((NC * NPAD, L), jnp.float32))

    scratch = [
        [pltpu.VMEM((C,), jnp.int32) for _ in range(NI)],
        [pltpu.VMEM((C,), jnp.int32) for _ in range(NI)],
        [pltpu.VMEM((C,), jnp.float32) for _ in range(NI)],
        [pltpu.VMEM((C, EMB), jnp.float32) for _ in range(NR)],
        [pltpu.SemaphoreType.DMA for _ in range(NI)],
        [pltpu.SemaphoreType.DMA for _ in range(NR)],
        [pltpu.SemaphoreType.DMA for _ in range(NR)],
        pltpu.VMEM_SHARED((NPAD, EMB), jnp.float32),
    ]
    if with_cnt:
        scratch += [
            pltpu.VMEM((C, L), jnp.float32),
            pltpu.VMEM((C, L), jnp.float32),
            pltpu.VMEM_SHARED((NPAD, L), jnp.float32),
            pltpu.SemaphoreType.DMA,
        ]

    @functools.partial(
        pl.kernel,
        mesh=mesh,
        out_type=out_type,
        scratch_types=scratch,
        compiler_params=pltpu.CompilerParams(use_tc_tiling_on_sc=False),
    )
    def sc_agg(table_h, eidx_h, w_h, *refs):
        if with_cnt:
            (featp_h, cntp_h, srcb, dstb, wb, rowsb, isem, gsem, ssem,
             accf, ones_v, z16_v, accc, csem) = refs
        else:
            (featp_h, srcb, dstb, wb, rowsb, isem, gsem, ssem, accf) = refs

        sid = lax.axis_index("s")
        cid = lax.axis_index("c")
        wid = cid * NS + sid
        base0 = wid * EPW

        def issue_idx(k, i):
            base = base0 + k * C
            pltpu.async_copy(eidx_h.at[pl.ds(base, C)], srcb[i], isem[i])
            pltpu.async_copy(eidx_h.at[pl.ds(E + base, C)], dstb[i], isem[i])
            pltpu.async_copy(w_h.at[pl.ds(base, C)], wb[i], isem[i])

        def wait_idx(i):
            pltpu.make_async_copy(eidx_h.at[pl.ds(0, C)], srcb[i], isem[i]).wait()
            pltpu.make_async_copy(eidx_h.at[pl.ds(0, C)], dstb[i], isem[i]).wait()
            pltpu.make_async_copy(w_h.at[pl.ds(0, C)], wb[i], isem[i]).wait()

        NSPLIT = 5
        CS = C // NSPLIT

        def issue_gather(i, b):
            for s in range(NSPLIT):
                pltpu.async_copy(
                    table_h.at[srcb[i].at[pl.ds(s * CS, CS)]],
                    rowsb[b].at[pl.ds(s * CS, CS)], gsem[b])

        def wait_gather(b):
            for s in range(NSPLIT):
                pltpu.make_async_copy(
                    table_h.at[srcb[0].at[pl.ds(0, CS)]],
                    rowsb[b].at[pl.ds(s * CS, CS)], gsem[b]).wait()

        def issue_scatter(i, b):
            pltpu.async_copy(rowsb[b], accf.at[dstb[i]], ssem[b], add=True)

        def wait_scatter(b):
            pltpu.make_async_copy(rowsb[b], accf.at[dstb[0]], ssem[b]).wait()

        def issue_cnt(i):
            pltpu.async_copy(ones_v, accc.at[dstb[i]], csem, add=True)

        def wait_cnt():
            pltpu.make_async_copy(ones_v, accc.at[dstb[0]], csem).wait()

        zv = jnp.zeros((L,), jnp.float32)
        ov = jnp.ones((L,), jnp.float32)

        def zero_body(i, _):
            for j in range(EMB // L):
                rowsb[0][i, pl.ds(j * L, L)] = zv
            if with_cnt:
                z16_v[i, :] = zv
                ones_v[i, :] = ov
            return 0

        lax.fori_loop(0, C, zero_body, 0)

        row0 = sid * ROWS_PT
        pltpu.sync_copy(rowsb[0].at[pl.ds(0, C)], accf.at[pl.ds(row0, C)])
        pltpu.sync_copy(rowsb[0].at[pl.ds(0, ROWS_PT - C)],
                        accf.at[pl.ds(row0 + C, ROWS_PT - C)])
        if with_cnt:
            pltpu.sync_copy(z16_v.at[pl.ds(0, C)], accc.at[pl.ds(row0, C)])
            pltpu.sync_copy(z16_v.at[pl.ds(0, ROWS_PT - C)],
                            accc.at[pl.ds(row0 + C, ROWS_PT - C)])

        issue_idx(0, 0)
        issue_idx(1, 1)
        wait_idx(0)
        issue_gather(0, 0)
        plsc.subcore_barrier()

        def body(k, phase, first):
            b = phase % NR
            o = (phase + 1) % NR
            i = phase % NI
            i1 = (phase + 1) % NI
            i2 = (phase + 2) % NI
            kt = k
            wait_gather(b)
            if not first:
                wait_scatter(o)

            @pl.when(kt <= NCHUNK - 2)
            def _():
                wait_idx(i1)
                issue_gather(i1, o)

            @pl.when(kt <= NCHUNK - 3)
            def _():
                issue_idx(k + 2, i2)

            if with_cnt:
                if not first:
                    wait_cnt()
                issue_cnt(i)
            _scale_rows_by_weight(rowsb[b], wb[i])
            issue_scatter(i, b)

        body(0, 0, first=True)

        def steady(t, _):
            k0 = 1 + t * 6
            for d in range(6):
                body(k0 + d, (1 + d) % 6, first=False)
            return 0

        lax.fori_loop(0, (NCHUNK - 1) // 6, steady, 0)

        wait_scatter((NCHUNK - 1) % NR)
        if with_cnt:
            wait_cnt()

        plsc.subcore_barrier()
        out0 = cid * NPAD + row0
        pltpu.sync_copy(accf.at[pl.ds(row0, ROWS_PT)],
                        featp_h.at[pl.ds(out0, ROWS_PT), pl.ds(0, EMB)])
        if with_cnt:
            pltpu.sync_copy(accc.at[pl.ds(row0, ROWS_PT)],
                            cntp_h.at[pl.ds(out0, ROWS_PT)])

    return sc_agg


_sc_agg_cnt = _make_sc_agg(with_cnt=True)
_sc_agg = _make_sc_agg(with_cnt=False)


def _tc_proj_body(x_ref, wl_ref, wr_ref, o1_ref, o2_ref):
    x = x_ref[...]
    o1_ref[...] = jnp.dot(x, wl_ref[...], preferred_element_type=jnp.float32)
    o2_ref[...] = jnp.dot(x, wr_ref[...], preferred_element_type=jnp.float32)


def _tc_proj(x, wlT, wrT):
    return pl.pallas_call(
        _tc_proj_body,
        out_shape=[
            jax.ShapeDtypeStruct((N, EMB), jnp.float32),
            jax.ShapeDtypeStruct((N, EMB), jnp.float32),
        ],
    )(x, wlT, wrT)


def _norm_relu(pre, gw, gb, gms):
    mean = jnp.mean(pre, axis=0, keepdims=True)
    cen = pre - gms * mean
    var = jnp.mean(cen * cen, axis=0, keepdims=True)
    return jnp.maximum(cen * jax.lax.rsqrt(var + 1e-5) * gw + gb, 0.0)


def _tc_mid(f1, c1, xr1, bl, gw, gb, gms, w2lT, w2rT):
    def body(f_ref, c_ref, xr_ref, bl_ref, gw_ref, gb_ref, gms_ref,
             w2l_ref, w2r_ref, h1_ref, hw_ref, hr_ref):
        cnt = c_ref[0:N, 0:1] + c_ref[NPAD:NPAD + N, 0:1]
        agg = ((f_ref[0:N, 0:EMB] + f_ref[NPAD:NPAD + N, 0:EMB])
               / jnp.maximum(cnt, 1.0))
        pre = agg + bl_ref[...] + xr_ref[...]
        h1 = _norm_relu(pre, gw_ref[...], gb_ref[...], gms_ref[...])
        h1_ref[...] = h1
        hw_ref[...] = jnp.dot(h1, w2l_ref[...], preferred_element_type=jnp.float32)
        hr_ref[...] = jnp.dot(h1, w2r_ref[...], preferred_element_type=jnp.float32)

    return pl.pallas_call(
        body,
        out_shape=[
            jax.ShapeDtypeStruct((N, EMB), jnp.float32),
            jax.ShapeDtypeStruct((N, EMB), jnp.float32),
            jax.ShapeDtypeStruct((N, EMB), jnp.float32),
        ],
    )(f1, c1, xr1, bl, gw, gb, gms, w2lT, w2rT)


def _sigmoid(x):
    return 1.0 / (1.0 + jnp.exp(-x))


def _tc_final(f2, c1, hr2, bl, gw, gb, gms, h1a,
              wih1T, bih1, bhh1, wih2T, bih2, bhh2):
    def body(f_ref, c_ref, hr_ref, bl_ref, gw_ref, gb_ref, gms_ref, h1a_ref,
             wih1_ref, bih1_ref, bhh1_ref, wih2_ref, bih2_ref, bhh2_ref,
             xc_ref, h1o_ref, h2o_ref):
        cnt = c_ref[0:N, 0:1] + c_ref[NPAD:NPAD + N, 0:1]
        agg = ((f_ref[0:N, 0:EMB] + f_ref[NPAD:NPAD + N, 0:EMB])
               / jnp.maximum(cnt, 1.0))
        pre = agg + bl_ref[...] + hr_ref[...]
        h2a = _norm_relu(pre, gw_ref[...], gb_ref[...], gms_ref[...])
        xc = jnp.concatenate([h1a_ref[...], h2a], axis=1)
        xc_ref[...] = xc

        gi = jnp.dot(xc, wih1_ref[...], preferred_element_type=jnp.float32) + bih1_ref[...]
        bhh = bhh1_ref[...]
        r = _sigmoid(gi[:, :EMB] + bhh[:, :EMB])
        z = _sigmoid(gi[:, EMB:2 * EMB] + bhh[:, EMB:2 * EMB])
        n = jnp.tanh(gi[:, 2 * EMB:] + r * bhh[:, 2 * EMB:])
        h1g = (1.0 - z) * n
        h1o_ref[...] = h1g

        gi2 = jnp.dot(h1g, wih2_ref[...], preferred_element_type=jnp.float32) + bih2_ref[...]
        bhh2v = bhh2_ref[...]
        r2 = _sigmoid(gi2[:, :EMB] + bhh2v[:, :EMB])
        z2 = _sigmoid(gi2[:, EMB:2 * EMB] + bhh2v[:, EMB:2 * EMB])
        n2 = jnp.tanh(gi2[:, 2 * EMB:] + r2 * bhh2v[:, 2 * EMB:])
        h2o_ref[...] = (1.0 - z2) * n2

    return pl.pallas_call(
        body,
        out_shape=[
            jax.ShapeDtypeStruct((N, 2 * EMB), jnp.float32),
            jax.ShapeDtypeStruct((N, EMB), jnp.float32),
            jax.ShapeDtypeStruct((N, EMB), jnp.float32),
        ],
    )(f2, c1, hr2, bl, gw, gb, gms, h1a, wih1T, bih1, bhh1, wih2T, bih2, bhh2)


def kernel(x, edge_index, edge_attr, W_l1, b_l1, W_r1, W_l2, b_l2, W_r2,
           gn1_w, gn1_b, gn1_ms, gn2_w, gn2_b, gn2_ms,
           W_ih1, W_hh1, b_ih1, b_hh1, W_ih2, W_hh2, b_ih2, b_hh2):
    eidx = edge_index.reshape(2 * E)
    w = edge_attr.reshape(E)

    xW1, xr1 = _tc_proj(x, W_l1.T, W_r1.T)

    f1, c1 = _sc_agg_cnt(xW1, eidx, w)

    h1a, hW2, hr2 = _tc_mid(
        f1, c1, xr1, b_l1.reshape(1, EMB),
        gn1_w.reshape(1, EMB), gn1_b.reshape(1, EMB), gn1_ms.reshape(1, EMB),
        W_l2.T, W_r2.T)

    (f2,) = _sc_agg(hW2, eidx, w)

    xc, h_1, h_2 = _tc_final(
        f2, c1, hr2, b_l2.reshape(1, EMB),
        gn2_w.reshape(1, EMB), gn2_b.reshape(1, EMB), gn2_ms.reshape(1, EMB),
        h1a, W_ih1.T, b_ih1.reshape(1, 3 * EMB), b_hh1.reshape(1, 3 * EMB),
        W_ih2.T, b_ih2.reshape(1, 3 * EMB), b_hh2.reshape(1, 3 * EMB))

    return (xc, h_1, h_2)
